# Initial kernel scaffold; baseline (speedup 1.0000x reference)
import jax
import jax.numpy as jnp
from jax.experimental import pallas as pl


def kernel(arm_loc_data, arm_conf_data, odm_loc_data, odm_conf_data, priors, targets):
    raise NotImplementedError("write your pallas kernel here")


# speedup vs baseline: 6.2105x; 6.2105x over previous
import functools

import jax
import jax.numpy as jnp
from jax import lax
from jax.experimental import pallas as pl
from jax.experimental.pallas import tpu as pltpu
from jax.experimental.pallas import tpu_sc as plsc

NUM_CLASSES = 21
THRESHOLD = 0.5
NEGPOS_RATIO = 3
VAR0, VAR1 = 0.1, 0.2

B = 32
P = 6375
O = 16
L = 16
PPAD = 6400
NG = PPAD // L
CHUNK_P = 640
NFULL = P // CHUNK_P
TAIL_P = P - NFULL * CHUNK_P

_LN2 = 0.6931471805599453
_SQRT2 = 1.4142135623730951


def _hsum(v):
  s = v[0]
  for i in range(1, L):
    s = s + v[i]
  return s


def _hmax(v):
  s = v[0]
  for i in range(1, L):
    s = jnp.maximum(s, v[i])
  return s


def _hmin(v):
  s = v[0]
  for i in range(1, L):
    s = jnp.minimum(s, v[i])
  return s


def _ln(x):
  b = plsc.bitcast(x, jnp.int32)
  e = lax.shift_right_logical(b, 23) - 127
  m = plsc.bitcast((b & 0x007FFFFF) | 0x3F800000, jnp.float32)
  big = m > _SQRT2
  m = jnp.where(big, m * 0.5, m)
  e = jnp.where(big, e + 1, e)
  z = (m - 1.0) / (m + 1.0)
  z2 = z * z
  p = 2.0 + z2 * (2.0 / 3.0 + z2 * (2.0 / 5.0 + z2 * (2.0 / 7.0 + z2 * (2.0 / 9.0))))
  return e.astype(jnp.float32) * _LN2 + z * p


def _body(conf_h, pt_h, lt_h, tg_h, out_h,
          pv, lv, tv, bto_r, bti_r, ct_r, vv_r, buf, res, sem):
  img = lax.axis_index("s") * 2 + lax.axis_index("c")
  iota = lax.iota(jnp.int32, L)

  pltpu.sync_copy(pt_h, pv)
  pltpu.sync_copy(lt_h.at[img], lv)
  pltpu.sync_copy(tg_h.at[img], tv)

  r0, r1, r2, r3 = tv[0], tv[1], tv[2], tv[3]
  t_x0 = [jnp.full((L,), r0[t]) for t in range(O)]
  t_y0 = [jnp.full((L,), r1[t]) for t in range(O)]
  t_x1 = [jnp.full((L,), r2[t]) for t in range(O)]
  t_y1 = [jnp.full((L,), r3[t]) for t in range(O)]
  t_ar = [(t_x1[t] - t_x0[t]) * (t_y1[t] - t_y0[t]) for t in range(O)]

  def g1(g, carry):
    bv, bi = carry
    base = g * L
    pidx = base + iota
    valid = pidx < P
    px = pv[0, pl.ds(base, L)]
    py = pv[1, pl.ds(base, L)]
    pw = pv[2, pl.ds(base, L)]
    ph = pv[3, pl.ds(base, L)]
    x0 = px - pw * 0.5
    x1 = px + pw * 0.5
    y0 = py - ph * 0.5
    y1 = py + ph * 0.5
    area_p = pw * ph
    bto_g = jnp.full((L,), -1.0)
    bti_g = jnp.zeros((L,), jnp.int32)
    nbv = []
    nbi = []
    for t in range(O):
      ix0 = jnp.maximum(x0, t_x0[t])
      ix1 = jnp.minimum(x1, t_x1[t])
      iy0 = jnp.maximum(y0, t_y0[t])
      iy1 = jnp.minimum(y1, t_y1[t])
      iw = jnp.maximum(ix1 - ix0, 0.0)
      ih = jnp.maximum(iy1 - iy0, 0.0)
      inter = iw * ih
      iou = inter / (t_ar[t] + area_p - inter)
      up = iou > bto_g
      bto_g = jnp.where(up, iou, bto_g)
      bti_g = jnp.where(up, t, bti_g)
      iou_m = jnp.where(valid, iou, -1.0)
      upt = iou_m > bv[t]
      nbv.append(jnp.where(upt, iou_m, bv[t]))
      nbi.append(jnp.where(upt, pidx, bi[t]))
    bto_r[pl.ds(base, L)] = jnp.where(valid, bto_g, 0.0)
    bti_r[pl.ds(base, L)] = bti_g
    return tuple(nbv), tuple(nbi)

  init = (tuple(jnp.full((L,), -2.0) for _ in range(O)),
          tuple(jnp.zeros((L,), jnp.int32) for _ in range(O)))
  bvf, bif = lax.fori_loop(0, NG, g1, init)

  bpiv = jnp.zeros((L,), jnp.int32)
  for t in range(O):
    m = _hmax(bvf[t])
    cand = jnp.where(bvf[t] == m, bif[t], jnp.int32(P))
    bpiv = jnp.where(iota == t, jnp.full((L,), _hmin(cand)), bpiv)

  plsc.store_scatter(bto_r, [bpiv], jnp.full((L,), 2.0))
  plsc.store_scatter(bti_r, [bpiv], iota)

  def g3(g, carry):
    ll, npv = carry
    base = g * L
    btog = bto_r[pl.ds(base, L)]
    btig = bti_r[pl.ds(base, L)]
    pos = btog >= THRESHOLD
    lab = plsc.load_gather(tv.at[4], [btig])
    ct = jnp.where(pos, lab.astype(jnp.int32) + 1, 0)
    ct_r[pl.ds(base, L)] = ct
    m0 = plsc.load_gather(tv.at[0], [btig])
    m1 = plsc.load_gather(tv.at[1], [btig])
    m2 = plsc.load_gather(tv.at[2], [btig])
    m3 = plsc.load_gather(tv.at[3], [btig])
    px = pv[0, pl.ds(base, L)]
    py = pv[1, pl.ds(base, L)]
    pw = pv[2, pl.ds(base, L)]
    ph = pv[3, pl.ds(base, L)]
    g0 = ((m0 + m2) * 0.5 - px) / (pw * VAR0)
    g1v = ((m1 + m3) * 0.5 - py) / (ph * VAR0)
    g2 = _ln((m2 - m0) / pw) * (1.0 / VAR1)
    g3v = _ln((m3 - m1) / ph) * (1.0 / VAR1)
    acc = jnp.zeros((L,))
    for c, gc in enumerate((g0, g1v, g2, g3v)):
      d = lv[c, pl.ds(base, L)] - gc
      ad = jnp.abs(d)
      acc = acc + jnp.where(ad < 1.0, 0.5 * d * d, ad - 0.5)
    ll = ll + jnp.where(pos, acc, 0.0)
    npv = npv + jnp.where(pos, 1, 0).astype(jnp.int32)
    return ll, npv

  ll, npv = lax.fori_loop(0, NG, g3, (jnp.zeros((L,)), jnp.zeros((L,), jnp.int32)))
  loss_l = _hsum(ll)
  npos = _hsum(npv)

  sp_acc = jnp.zeros((L,))
  for ci in range(NFULL + 1):
    pstart = ci * CHUNK_P
    cnt = CHUNK_P if ci < NFULL else TAIL_P
    dst = buf if ci < NFULL else buf.at[pl.ds(0, TAIL_P)]
    pltpu.sync_copy(conf_h.at[img, pl.ds(pstart, cnt)], dst)
    ngr = (cnt + L - 1) // L

    def g4(g, acc, pstart=pstart):
      base_l = g * L
      base_g = pstart + base_l
      pidx = base_g + iota
      valid = pidx < P
      ip = base_l + iota
      xs = [plsc.load_gather(buf, [ip, jnp.full((L,), j, jnp.int32)])
            for j in range(NUM_CLASSES)]
      m = xs[0]
      for j in range(1, NUM_CLASSES):
        m = jnp.maximum(m, xs[j])
      s = jnp.zeros((L,))
      for j in range(NUM_CLASSES):
        s = s + jnp.exp(xs[j] - m)
      lse = _ln(s) + m
      ctg = ct_r[pl.ds(base_g, L)]
      tgt = plsc.load_gather(buf, [ip, ctg])
      ce = lse - tgt
      pos = bto_r[pl.ds(base_g, L)] >= THRESHOLD
      acc = acc + jnp.where(pos & valid, ce, 0.0)
      vv_r[pl.ds(base_g, L)] = jnp.where(pos | (~valid), 0.0, ce)
      return acc

    sp_acc = lax.fori_loop(0, ngr, g4, sp_acc)
  vv_r[pl.ds(NFULL * CHUNK_P + ((TAIL_P + L - 1) // L) * L,
             PPAD - NFULL * CHUNK_P - ((TAIL_P + L - 1) // L) * L)] = (
      jnp.zeros((PPAD - NFULL * CHUNK_P - ((TAIL_P + L - 1) // L) * L,)))

  k = jnp.minimum(jnp.int32(NEGPOS_RATIO) * npos, jnp.int32(P - 1))

  def count_ge(th):
    thv = jnp.full((L,), th)

    def gb(g, acc):
      vb = plsc.bitcast(vv_r[pl.ds(g * L, L)], jnp.int32)
      return acc + jnp.where(vb >= thv, 1, 0).astype(jnp.int32)

    return _hsum(lax.fori_loop(0, NG, gb, jnp.zeros((L,), jnp.int32)))

  def bs(_, lohi):
    lo, hi = lohi
    mid = lo + lax.shift_right_logical(hi - lo, 1)
    ok = count_ge(mid) >= k
    return jnp.where(ok, mid, lo), jnp.where(ok, hi, mid)

  lo, hi = lax.fori_loop(0, 31, bs, (jnp.int32(0), jnp.int32(0x7F800000)))
  bstar = jnp.full((L,), lo)

  def gf(g, carry):
    sh, ch = carry
    v = vv_r[pl.ds(g * L, L)]
    vb = plsc.bitcast(v, jnp.int32)
    gt = vb > bstar
    sh = sh + jnp.where(gt, v, 0.0)
    ch = ch + jnp.where(gt, 1, 0).astype(jnp.int32)
    return sh, ch

  sh, ch = lax.fori_loop(0, NG, gf, (jnp.zeros((L,)), jnp.zeros((L,), jnp.int32)))
  sum_hi = _hsum(sh)
  c_hi = _hsum(ch)
  tie_val = plsc.bitcast(bstar, jnp.float32)[0]
  loss_c = _hsum(sp_acc) + sum_hi + (k - c_hi).astype(jnp.float32) * tie_val

  outv = jnp.where(iota == 0, jnp.full((L,), loss_l),
                   jnp.where(iota == 1, jnp.full((L,), loss_c),
                             jnp.where(iota == 2,
                                       jnp.full((L,), npos.astype(jnp.float32)),
                                       0.0)))
  res[...] = outv
  pltpu.sync_copy(res, out_h.at[img])


@jax.jit
def _run(conf, pt, lt, tg):
  mesh = plsc.VectorSubcoreMesh(core_axis_name="c", subcore_axis_name="s",
                                num_cores=2, num_subcores=16)
  f = pl.kernel(
      _body,
      out_type=jax.ShapeDtypeStruct((B, L), jnp.float32),
      mesh=mesh,
      compiler_params=pltpu.CompilerParams(needs_layout_passes=False,
                                           use_tc_tiling_on_sc=False),
      scratch_types=[
          pltpu.VMEM((4, PPAD), jnp.float32),
          pltpu.VMEM((4, PPAD), jnp.float32),
          pltpu.VMEM((5, O), jnp.float32),
          pltpu.VMEM((PPAD,), jnp.float32),
          pltpu.VMEM((PPAD,), jnp.int32),
          pltpu.VMEM((PPAD,), jnp.int32),
          pltpu.VMEM((PPAD,), jnp.float32),
          pltpu.VMEM((CHUNK_P, NUM_CLASSES), jnp.float32),
          pltpu.VMEM((L,), jnp.float32),
          pltpu.SemaphoreType.DMA,
      ],
  )
  return f(conf, pt, lt, tg)


def kernel(arm_loc_data, arm_conf_data, odm_loc_data, odm_conf_data,
           priors, targets):
  del odm_loc_data, odm_conf_data
  pt = jnp.pad(priors.T, ((0, 0), (0, PPAD - P)), constant_values=0.5)
  lt = jnp.pad(jnp.transpose(arm_loc_data, (0, 2, 1)),
               ((0, 0), (0, 0), (0, PPAD - P)))
  tg = jnp.transpose(targets, (0, 2, 1))
  out = _run(arm_conf_data, pt, lt, tg)
  loss_l = jnp.sum(out[:, 0])
  loss_c = jnp.sum(out[:, 1])
  n = jnp.sum(out[:, 2])
  return (loss_l / n, loss_c / n)


# speedup vs baseline: 7.6336x; 1.2292x over previous
import functools

import jax
import jax.numpy as jnp
from jax import lax
from jax.experimental import pallas as pl
from jax.experimental.pallas import tpu as pltpu
from jax.experimental.pallas import tpu_sc as plsc

NUM_CLASSES = 21
THRESHOLD = 0.5
NEGPOS_RATIO = 3
VAR0, VAR1 = 0.1, 0.2

B = 32
P = 6375
O = 16
L = 16
PPAD = 6400
NG = PPAD // L
CHUNK_P = 640
NFULL = P // CHUNK_P
TAIL_P = P - NFULL * CHUNK_P

_LN2 = 0.6931471805599453
_SQRT2 = 1.4142135623730951


def _hsum(v):
  s = v[0]
  for i in range(1, L):
    s = s + v[i]
  return s


def _hmax(v):
  s = v[0]
  for i in range(1, L):
    s = jnp.maximum(s, v[i])
  return s


def _hmin(v):
  s = v[0]
  for i in range(1, L):
    s = jnp.minimum(s, v[i])
  return s


def _ln(x):
  b = plsc.bitcast(x, jnp.int32)
  e = lax.shift_right_logical(b, 23) - 127
  m = plsc.bitcast((b & 0x007FFFFF) | 0x3F800000, jnp.float32)
  big = m > _SQRT2
  m = jnp.where(big, m * 0.5, m)
  e = jnp.where(big, e + 1, e)
  z = (m - 1.0) / (m + 1.0)
  z2 = z * z
  p = 2.0 + z2 * (2.0 / 3.0 + z2 * (2.0 / 5.0 + z2 * (2.0 / 7.0 + z2 * (2.0 / 9.0))))
  return e.astype(jnp.float32) * _LN2 + z * p


def _body(conf_h, loc_h, pri_h, tgt_h, out_h,
          pv, lv, tv, bto_r, bti_r, ct_r, vv_r, buf, res, sem):
  img = lax.axis_index("s") * 2 + lax.axis_index("c")
  iota = lax.iota(jnp.int32, L)

  pltpu.sync_copy(pri_h, pv)
  pltpu.sync_copy(loc_h.at[img], lv)
  pltpu.sync_copy(tgt_h.at[img], tv)

  def col(ref, base_idx, c):
    return plsc.load_gather(ref, [base_idx + c])

  ti5 = iota * 5
  r0 = col(tv, ti5, 0)
  r1 = col(tv, ti5, 1)
  r2 = col(tv, ti5, 2)
  r3 = col(tv, ti5, 3)
  t_x0 = [jnp.full((L,), r0[t]) for t in range(O)]
  t_y0 = [jnp.full((L,), r1[t]) for t in range(O)]
  t_x1 = [jnp.full((L,), r2[t]) for t in range(O)]
  t_y1 = [jnp.full((L,), r3[t]) for t in range(O)]
  t_ar = [(t_x1[t] - t_x0[t]) * (t_y1[t] - t_y0[t]) for t in range(O)]

  def g1(g, carry):
    bv, bi = carry
    base = g * L
    pidx = base + iota
    valid = pidx < P
    pidc4 = jnp.minimum(pidx, P - 1) * 4
    px = col(pv, pidc4, 0)
    py = col(pv, pidc4, 1)
    pw = col(pv, pidc4, 2)
    ph = col(pv, pidc4, 3)
    x0 = px - pw * 0.5
    x1 = px + pw * 0.5
    y0 = py - ph * 0.5
    y1 = py + ph * 0.5
    area_p = pw * ph
    bto_g = jnp.full((L,), -1.0)
    bti_g = jnp.zeros((L,), jnp.int32)
    nbv = []
    nbi = []
    for t in range(O):
      ix0 = jnp.maximum(x0, t_x0[t])
      ix1 = jnp.minimum(x1, t_x1[t])
      iy0 = jnp.maximum(y0, t_y0[t])
      iy1 = jnp.minimum(y1, t_y1[t])
      iw = jnp.maximum(ix1 - ix0, 0.0)
      ih = jnp.maximum(iy1 - iy0, 0.0)
      inter = iw * ih
      iou = inter / (t_ar[t] + area_p - inter)
      up = iou > bto_g
      bto_g = jnp.where(up, iou, bto_g)
      bti_g = jnp.where(up, t, bti_g)
      iou_m = jnp.where(valid, iou, -1.0)
      upt = iou_m > bv[t]
      nbv.append(jnp.where(upt, iou_m, bv[t]))
      nbi.append(jnp.where(upt, pidx, bi[t]))
    bto_r[pl.ds(base, L)] = jnp.where(valid, bto_g, 0.0)
    bti_r[pl.ds(base, L)] = bti_g
    return tuple(nbv), tuple(nbi)

  init = (tuple(jnp.full((L,), -2.0) for _ in range(O)),
          tuple(jnp.zeros((L,), jnp.int32) for _ in range(O)))
  bvf, bif = lax.fori_loop(0, NG, g1, init)

  bpiv = jnp.zeros((L,), jnp.int32)
  for t in range(O):
    m = _hmax(bvf[t])
    cand = jnp.where(bvf[t] == m, bif[t], jnp.int32(P))
    bpiv = jnp.where(iota == t, jnp.full((L,), _hmin(cand)), bpiv)

  plsc.store_scatter(bto_r, [bpiv], jnp.full((L,), 2.0))
  plsc.store_scatter(bti_r, [bpiv], iota)

  def g3(g, carry):
    ll, npv = carry
    base = g * L
    pidx = base + iota
    pidc4 = jnp.minimum(pidx, P - 1) * 4
    btog = bto_r[pl.ds(base, L)]
    btig = bti_r[pl.ds(base, L)]
    pos = btog >= THRESHOLD
    bt5 = btig * 5
    lab = col(tv, bt5, 4)
    ct = jnp.where(pos, lab.astype(jnp.int32) + 1, 0)
    ct_r[pl.ds(base, L)] = ct
    m0 = col(tv, bt5, 0)
    m1 = col(tv, bt5, 1)
    m2 = col(tv, bt5, 2)
    m3 = col(tv, bt5, 3)
    px = col(pv, pidc4, 0)
    py = col(pv, pidc4, 1)
    pw = col(pv, pidc4, 2)
    ph = col(pv, pidc4, 3)
    g0 = ((m0 + m2) * 0.5 - px) / (pw * VAR0)
    g1v = ((m1 + m3) * 0.5 - py) / (ph * VAR0)
    g2 = _ln((m2 - m0) / pw) * (1.0 / VAR1)
    g3v = _ln((m3 - m1) / ph) * (1.0 / VAR1)
    acc = jnp.zeros((L,))
    for c, gc in enumerate((g0, g1v, g2, g3v)):
      d = col(lv, pidc4, c) - gc
      ad = jnp.abs(d)
      acc = acc + jnp.where(ad < 1.0, 0.5 * d * d, ad - 0.5)
    ll = ll + jnp.where(pos, acc, 0.0)
    npv = npv + jnp.where(pos, 1, 0).astype(jnp.int32)
    return ll, npv

  ll, npv = lax.fori_loop(0, NG, g3, (jnp.zeros((L,)), jnp.zeros((L,), jnp.int32)))
  loss_l = _hsum(ll)
  npos = _hsum(npv)

  sp_acc = jnp.zeros((L,))
  for ci in range(NFULL + 1):
    pstart = ci * CHUNK_P
    cnt = CHUNK_P if ci < NFULL else TAIL_P
    dst = buf if ci < NFULL else buf.at[pl.ds(0, TAIL_P * NUM_CLASSES)]
    pltpu.sync_copy(
        conf_h.at[img, pl.ds(pstart * NUM_CLASSES, cnt * NUM_CLASSES)], dst)
    ngr = (cnt + L - 1) // L

    def g4(g, acc, pstart=pstart):
      base_l = g * L
      base_g = pstart + base_l
      pidx = base_g + iota
      valid = pidx < P
      fidx = (base_l + iota) * NUM_CLASSES
      xs = [plsc.load_gather(buf, [fidx + j]) for j in range(NUM_CLASSES)]
      m = xs[0]
      for j in range(1, NUM_CLASSES):
        m = jnp.maximum(m, xs[j])
      s = jnp.zeros((L,))
      for j in range(NUM_CLASSES):
        s = s + jnp.exp(xs[j] - m)
      lse = _ln(s) + m
      ctg = ct_r[pl.ds(base_g, L)]
      tgt = plsc.load_gather(buf, [fidx + ctg])
      ce = lse - tgt
      pos = bto_r[pl.ds(base_g, L)] >= THRESHOLD
      acc = acc + jnp.where(pos & valid, ce, 0.0)
      vv_r[pl.ds(base_g, L)] = jnp.where(pos | (~valid), 0.0, ce)
      return acc

    sp_acc = lax.fori_loop(0, ngr, g4, sp_acc)
  vv_r[pl.ds(NFULL * CHUNK_P + ((TAIL_P + L - 1) // L) * L,
             PPAD - NFULL * CHUNK_P - ((TAIL_P + L - 1) // L) * L)] = (
      jnp.zeros((PPAD - NFULL * CHUNK_P - ((TAIL_P + L - 1) // L) * L,)))

  k = jnp.minimum(jnp.int32(NEGPOS_RATIO) * npos, jnp.int32(P - 1))

  def count_ge(th):
    thv = jnp.full((L,), th)

    def gb(g, acc):
      vb = plsc.bitcast(vv_r[pl.ds(g * L, L)], jnp.int32)
      return acc + jnp.where(vb >= thv, 1, 0).astype(jnp.int32)

    return _hsum(lax.fori_loop(0, NG, gb, jnp.zeros((L,), jnp.int32)))

  def bs(_, lohi):
    lo, hi = lohi
    mid = lo + lax.shift_right_logical(hi - lo, 1)
    ok = count_ge(mid) >= k
    return jnp.where(ok, mid, lo), jnp.where(ok, hi, mid)

  lo, hi = lax.fori_loop(0, 31, bs, (jnp.int32(0), jnp.int32(0x7F800000)))
  bstar = jnp.full((L,), lo)

  def gf(g, carry):
    sh, ch = carry
    v = vv_r[pl.ds(g * L, L)]
    vb = plsc.bitcast(v, jnp.int32)
    gt = vb > bstar
    sh = sh + jnp.where(gt, v, 0.0)
    ch = ch + jnp.where(gt, 1, 0).astype(jnp.int32)
    return sh, ch

  sh, ch = lax.fori_loop(0, NG, gf, (jnp.zeros((L,)), jnp.zeros((L,), jnp.int32)))
  sum_hi = _hsum(sh)
  c_hi = _hsum(ch)
  tie_val = plsc.bitcast(bstar, jnp.float32)[0]
  loss_c = _hsum(sp_acc) + sum_hi + (k - c_hi).astype(jnp.float32) * tie_val

  outv = jnp.where(iota == 0, jnp.full((L,), loss_l),
                   jnp.where(iota == 1, jnp.full((L,), loss_c),
                             jnp.where(iota == 2,
                                       jnp.full((L,), npos.astype(jnp.float32)),
                                       0.0)))
  res[...] = outv
  pltpu.sync_copy(res, out_h.at[img])


@jax.jit
def _run(conf, loc, pri, tgt):
  mesh = plsc.VectorSubcoreMesh(core_axis_name="c", subcore_axis_name="s",
                                num_cores=2, num_subcores=16)
  f = pl.kernel(
      _body,
      out_type=jax.ShapeDtypeStruct((B, L), jnp.float32),
      mesh=mesh,
      compiler_params=pltpu.CompilerParams(needs_layout_passes=False,
                                           use_tc_tiling_on_sc=False),
      scratch_types=[
          pltpu.VMEM((P * 4,), jnp.float32),
          pltpu.VMEM((P * 4,), jnp.float32),
          pltpu.VMEM((O * 5,), jnp.float32),
          pltpu.VMEM((PPAD,), jnp.float32),
          pltpu.VMEM((PPAD,), jnp.int32),
          pltpu.VMEM((PPAD,), jnp.int32),
          pltpu.VMEM((PPAD,), jnp.float32),
          pltpu.VMEM((CHUNK_P * NUM_CLASSES,), jnp.float32),
          pltpu.VMEM((L,), jnp.float32),
          pltpu.SemaphoreType.DMA,
      ],
  )
  return f(conf, loc, pri, tgt)


def kernel(arm_loc_data, arm_conf_data, odm_loc_data, odm_conf_data,
           priors, targets):
  del odm_loc_data, odm_conf_data
  out = _run(arm_conf_data.reshape(B, P * NUM_CLASSES),
             arm_loc_data.reshape(B, P * 4),
             priors.reshape(P * 4),
             targets.reshape(B, O * 5))
  loss_l = jnp.sum(out[:, 0])
  loss_c = jnp.sum(out[:, 1])
  n = jnp.sum(out[:, 2])
  return (loss_l / n, loss_c / n)
